# Initial kernel scaffold; baseline (speedup 1.0000x reference)
import jax
import jax.numpy as jnp
from jax.experimental import pallas as pl


def kernel(x, edge_index, W1, b1, Wmu, bmu, Wls, bls):
    raise NotImplementedError("write your pallas kernel here")


# speedup vs baseline: 15.9391x; 15.9391x over previous
import functools

import jax
import jax.numpy as jnp
from jax import lax
from jax.experimental import pallas as pl
from jax.experimental.pallas import tpu as pltpu
from jax.experimental.pallas import tpu_sc as plsc

N = 10000
E = 160000
D_IN = 256
D_HID = 128
D_OUT = 64

NP = 10240
NC = 2
NS = 16
CH = 128
EC = E // NC
CPC = EC // CH
JMAX = (CPC + NS - 1) // NS
RPT = NP // NS

_mesh = plsc.VectorSubcoreMesh(core_axis_name="c", subcore_axis_name="s")


@functools.partial(
    pl.kernel,
    mesh=_mesh,
    out_type=jax.ShapeDtypeStruct((NC * NP,), jnp.float32),
    scratch_types=[
        pltpu.VMEM((CH,), jnp.int32),
        pltpu.VMEM((CH,), jnp.float32),
        pltpu.VMEM((RPT,), jnp.float32),
        pltpu.VMEM_SHARED((NP,), jnp.float32),
    ],
)
def _deg_kernel(dst_hbm, out_hbm, dst_v, ones_v, zero_v, acc):
    c = lax.axis_index("c")
    s = lax.axis_index("s")
    for i in range(RPT // 16):
        zero_v[pl.ds(i * 16, 16)] = jnp.zeros((16,), jnp.float32)
    for i in range(CH // 16):
        ones_v[pl.ds(i * 16, 16)] = jnp.ones((16,), jnp.float32)
    rbase = s * RPT
    pltpu.sync_copy(zero_v, acc.at[pl.ds(rbase, RPT)])
    plsc.subcore_barrier()

    def body(j, carry):
        cid = j * NS + s

        @pl.when(cid < CPC)
        def _():
            base = c * EC + cid * CH
            pltpu.sync_copy(dst_hbm.at[pl.ds(base, CH)], dst_v)
            pltpu.sync_copy(ones_v, acc.at[dst_v], add=True)

        return carry

    lax.fori_loop(0, JMAX, body, 0)
    plsc.subcore_barrier()
    pltpu.sync_copy(acc.at[pl.ds(rbase, RPT)],
                    out_hbm.at[pl.ds(c * NP + rbase, RPT)])


@functools.partial(
    pl.kernel,
    mesh=_mesh,
    out_type=jax.ShapeDtypeStruct((NC * NP, D_HID), jnp.float32),
    scratch_types=[
        pltpu.VMEM((CH,), jnp.int32),
        pltpu.VMEM((CH,), jnp.int32),
        pltpu.VMEM((CH, D_HID), jnp.float32),
        pltpu.VMEM_SHARED((NP, D_HID), jnp.float32),
        pltpu.SemaphoreType.DMA,
    ],
)
def _scatter_kernel(g_hbm, src_hbm, dst_hbm, out_hbm,
                    src_v, dst_v, rows_v, acc, sem):
    c = lax.axis_index("c")
    s = lax.axis_index("s")
    rbase = s * RPT
    pltpu.sync_copy(g_hbm.at[pl.ds(rbase, RPT)], acc.at[pl.ds(rbase, RPT)])
    plsc.subcore_barrier()

    def body(j, carry):
        cid = j * NS + s

        @pl.when(cid < CPC)
        def _():
            base = c * EC + cid * CH
            pltpu.sync_copy(src_hbm.at[pl.ds(base, CH)], src_v)
            pltpu.sync_copy(dst_hbm.at[pl.ds(base, CH)], dst_v)
            pltpu.async_copy(g_hbm.at[src_v], rows_v, sem).wait()
            pltpu.sync_copy(rows_v, acc.at[dst_v], add=True)

        return carry

    lax.fori_loop(0, JMAX, body, 0)
    plsc.subcore_barrier()
    pltpu.sync_copy(acc.at[pl.ds(rbase, RPT)],
                    out_hbm.at[pl.ds(c * NP + rbase, RPT)])


BN = 1024


def _mm1_body(x_ref, w_ref, dinv_ref, g_ref):
    h = jnp.dot(x_ref[...], w_ref[...], preferred_element_type=jnp.float32)
    g_ref[...] = h * dinv_ref[...]


def _mm2_body(s_ref, g1_ref, dinv_ref, b1_ref, w_ref, g2_ref):
    agg = s_ref[0] + s_ref[1] - g1_ref[...]
    h = jnp.maximum(dinv_ref[...] * agg + b1_ref[...], 0.0)
    h2 = jnp.dot(h, w_ref[...], preferred_element_type=jnp.float32)
    g2_ref[...] = h2 * dinv_ref[...]


def _fin_body(t_ref, g2_ref, dinv_ref, bc_ref, o_ref):
    agg = t_ref[0] + t_ref[1] - g2_ref[...]
    o_ref[...] = dinv_ref[...] * agg + bc_ref[...]


def _col_spec():
    return pl.BlockSpec((BN, 1), lambda i: (i, 0))


def _row_spec(d):
    return pl.BlockSpec((BN, d), lambda i: (i, 0))


def _full_spec(r, d):
    return pl.BlockSpec((r, d), lambda i: (0, 0))


def _pair_spec(d):
    return pl.BlockSpec((NC, BN, d), lambda i: (0, i, 0))


def kernel(x, edge_index, W1, b1, Wmu, bmu, Wls, bls):
    src = edge_index[0]
    dst = edge_index[1]
    xp = jnp.pad(x, ((0, NP - N), (0, 0)))
    Wcat = jnp.concatenate([Wmu, Wls], axis=1)
    bcat = jnp.concatenate([bmu, bls]).reshape(1, 2 * D_OUT)
    b1r = b1.reshape(1, D_HID)

    degp = _deg_kernel(dst)
    deg = 1.0 + degp[:NP] + degp[NP:]
    dinv = lax.rsqrt(deg).reshape(NP, 1)

    grid = NP // BN

    g1 = pl.pallas_call(
        _mm1_body,
        grid=(grid,),
        in_specs=[_row_spec(D_IN), _full_spec(D_IN, D_HID), _col_spec()],
        out_specs=_row_spec(D_HID),
        out_shape=jax.ShapeDtypeStruct((NP, D_HID), jnp.float32),
    )(xp, W1, dinv)

    s_pair = _scatter_kernel(g1, src, dst).reshape(NC, NP, D_HID)

    g2 = pl.pallas_call(
        _mm2_body,
        grid=(grid,),
        in_specs=[_pair_spec(D_HID), _row_spec(D_HID), _col_spec(),
                  _full_spec(1, D_HID), _full_spec(D_HID, D_HID)],
        out_specs=_row_spec(D_HID),
        out_shape=jax.ShapeDtypeStruct((NP, D_HID), jnp.float32),
    )(s_pair, g1, dinv, b1r, Wcat)

    t_pair = _scatter_kernel(g2, src, dst).reshape(NC, NP, D_HID)

    out = pl.pallas_call(
        _fin_body,
        grid=(grid,),
        in_specs=[_pair_spec(D_HID), _row_spec(D_HID), _col_spec(),
                  _full_spec(1, D_HID)],
        out_specs=_row_spec(D_HID),
        out_shape=jax.ShapeDtypeStruct((NP, D_HID), jnp.float32),
    )(t_pair, g2, dinv, bcat)

    out = out[:N]
    return (out[:, :D_OUT], out[:, D_OUT:])


# speedup vs baseline: 25.9258x; 1.6266x over previous
import functools

import jax
import jax.numpy as jnp
from jax import lax
from jax.experimental import pallas as pl
from jax.experimental.pallas import tpu as pltpu
from jax.experimental.pallas import tpu_sc as plsc

N = 10000
E = 160000
D_IN = 256
D_HID = 128
D_OUT = 64

NP = 10240
NC = 2
NS = 16
CH = 128
EC = E // NC
CPC = EC // CH
NCHT = CPC // NS
RPT = NP // NS

_mesh = plsc.VectorSubcoreMesh(core_axis_name="c", subcore_axis_name="s")


@functools.partial(
    pl.kernel,
    mesh=_mesh,
    out_type=jax.ShapeDtypeStruct((NC * NP,), jnp.float32),
    scratch_types=[
        pltpu.VMEM((NCHT + 1, 2, CH), jnp.int32),
        pltpu.VMEM((CH,), jnp.float32),
        pltpu.VMEM((RPT,), jnp.float32),
        pltpu.VMEM_SHARED((NP,), jnp.float32),
        pltpu.SemaphoreType.DMA,
        pltpu.SemaphoreType.DMA,
    ],
)
def _deg_kernel(ep_hbm, out_hbm, idx_all, ones_v, zero_v, acc, d0, d1):
    c = lax.axis_index("c")
    s = lax.axis_index("s")
    tc0 = c * CPC + s * NCHT
    rbase = s * RPT
    idx_cp = pltpu.async_copy(ep_hbm.at[pl.ds(tc0, NCHT)],
                              idx_all.at[pl.ds(0, NCHT)], d0)
    for i in range(RPT // 16):
        zero_v[pl.ds(i * 16, 16)] = jnp.zeros((16,), jnp.float32)
    for i in range(CH // 16):
        ones_v[pl.ds(i * 16, 16)] = jnp.ones((16,), jnp.float32)
    pltpu.sync_copy(zero_v, acc.at[pl.ds(rbase, RPT)])
    idx_cp.wait()

    @pl.when(s == 0)
    def _():
        pltpu.sync_copy(ep_hbm.at[pl.ds(c * CPC + NS * NCHT, 1)],
                        idx_all.at[pl.ds(NCHT, 1)])

    plsc.subcore_barrier()

    def sstart(ci, sem):
        pltpu.async_copy(ones_v, acc.at[idx_all.at[ci, 1]], sem, add=True)

    def swait(sem):
        pltpu.make_async_copy(ones_v, acc.at[idx_all.at[0, 1]], sem).wait()

    sstart(0, d0)
    sstart(1, d1)

    def body(j, carry):
        swait(d0)
        sstart(2 * j + 2, d0)
        swait(d1)
        sstart(2 * j + 3, d1)
        return carry

    lax.fori_loop(0, NCHT // 2 - 1, body, 0)
    swait(d0)
    sstart(NCHT - 1, d0)

    @pl.when(s == 0)
    def _():
        swait(d1)
        sstart(NCHT, d1)
        swait(d1)

    @pl.when(s != 0)
    def _():
        swait(d1)

    swait(d0)
    plsc.subcore_barrier()
    pltpu.sync_copy(acc.at[pl.ds(rbase, RPT)],
                    out_hbm.at[pl.ds(c * NP + rbase, RPT)])


@functools.partial(
    pl.kernel,
    mesh=_mesh,
    out_type=jax.ShapeDtypeStruct((NC * NP, D_HID), jnp.float32),
    scratch_types=[
        pltpu.VMEM((NCHT + 1, 2, CH), jnp.int32),
        pltpu.VMEM((CH, D_HID), jnp.float32),
        pltpu.VMEM((CH, D_HID), jnp.float32),
        pltpu.VMEM_SHARED((NP, D_HID), jnp.float32),
        pltpu.SemaphoreType.DMA,
        pltpu.SemaphoreType.DMA,
        pltpu.SemaphoreType.DMA,
        pltpu.SemaphoreType.DMA,
    ],
)
def _scatter_kernel(g_hbm, ep_hbm, out_hbm, idx_all, rows0, rows1, acc,
                    sg0, sg1, ss0, ss1):
    c = lax.axis_index("c")
    s = lax.axis_index("s")
    tc0 = c * CPC + s * NCHT
    rbase = s * RPT
    idx_cp = pltpu.async_copy(ep_hbm.at[pl.ds(tc0, NCHT)],
                              idx_all.at[pl.ds(0, NCHT)], sg1)
    pltpu.sync_copy(g_hbm.at[pl.ds(rbase, RPT)], acc.at[pl.ds(rbase, RPT)])
    idx_cp.wait()

    @pl.when(s == 0)
    def _():
        pltpu.sync_copy(ep_hbm.at[pl.ds(c * CPC + NS * NCHT, 1)],
                        idx_all.at[pl.ds(NCHT, 1)])

    plsc.subcore_barrier()

    def gstart(rows, ci, sem):
        pltpu.async_copy(g_hbm.at[idx_all.at[ci, 0]], rows, sem)

    def gwait(rows, sem):
        pltpu.make_async_copy(g_hbm.at[idx_all.at[0, 0]], rows, sem).wait()

    def sstart(rows, ci, sem):
        pltpu.async_copy(rows, acc.at[idx_all.at[ci, 1]], sem, add=True)

    def swait(rows, sem):
        pltpu.make_async_copy(rows, acc.at[idx_all.at[0, 1]], sem).wait()

    gstart(rows0, 0, sg0)

    def body(j, carry):
        c0 = 2 * j
        c1 = 2 * j + 1

        @pl.when(j > 0)
        def _():
            swait(rows1, ss1)

        gwait(rows0, sg0)
        gstart(rows1, c1, sg1)
        sstart(rows0, c0, ss0)
        gwait(rows1, sg1)
        swait(rows0, ss0)
        gstart(rows0, c0 + 2, sg0)
        sstart(rows1, c1, ss1)
        return carry

    lax.fori_loop(0, (NCHT - 1) // 2, body, 0)
    swait(rows1, ss1)
    gwait(rows0, sg0)
    sstart(rows0, NCHT - 1, ss0)

    @pl.when(s == 0)
    def _():
        gstart(rows1, NCHT, sg1)
        gwait(rows1, sg1)
        sstart(rows1, NCHT, ss1)
        swait(rows1, ss1)

    swait(rows0, ss0)
    plsc.subcore_barrier()
    pltpu.sync_copy(acc.at[pl.ds(rbase, RPT)],
                    out_hbm.at[pl.ds(c * NP + rbase, RPT)])


BN = 1024


def _mm1_body(x_ref, w_ref, dinv_ref, g_ref):
    h = jnp.dot(x_ref[...], w_ref[...], preferred_element_type=jnp.float32)
    g_ref[...] = h * dinv_ref[...]


def _mm2_body(s_ref, g1_ref, dinv_ref, b1_ref, w_ref, g2_ref):
    agg = s_ref[0] + s_ref[1] - g1_ref[...]
    h = jnp.maximum(dinv_ref[...] * agg + b1_ref[...], 0.0)
    h2 = jnp.dot(h, w_ref[...], preferred_element_type=jnp.float32)
    g2_ref[...] = h2 * dinv_ref[...]


def _fin_body(t_ref, g2_ref, dinv_ref, bc_ref, o_ref):
    agg = t_ref[0] + t_ref[1] - g2_ref[...]
    o_ref[...] = dinv_ref[...] * agg + bc_ref[...]


def _col_spec():
    return pl.BlockSpec((BN, 1), lambda i: (i, 0))


def _row_spec(d):
    return pl.BlockSpec((BN, d), lambda i: (i, 0))


def _full_spec(r, d):
    return pl.BlockSpec((r, d), lambda i: (0, 0))


def _pair_spec(d):
    return pl.BlockSpec((NC, BN, d), lambda i: (0, i, 0))


def kernel(x, edge_index, W1, b1, Wmu, bmu, Wls, bls):
    ep = edge_index.reshape(2, E // CH, CH).swapaxes(0, 1)
    xp = jnp.pad(x, ((0, NP - N), (0, 0)))
    Wcat = jnp.concatenate([Wmu, Wls], axis=1)
    bcat = jnp.concatenate([bmu, bls]).reshape(1, 2 * D_OUT)
    b1r = b1.reshape(1, D_HID)

    degp = _deg_kernel(ep)
    deg = 1.0 + degp[:NP] + degp[NP:]
    dinv = lax.rsqrt(deg).reshape(NP, 1)

    grid = NP // BN

    g1 = pl.pallas_call(
        _mm1_body,
        grid=(grid,),
        in_specs=[_row_spec(D_IN), _full_spec(D_IN, D_HID), _col_spec()],
        out_specs=_row_spec(D_HID),
        out_shape=jax.ShapeDtypeStruct((NP, D_HID), jnp.float32),
    )(xp, W1, dinv)

    s_pair = _scatter_kernel(g1, ep).reshape(NC, NP, D_HID)

    g2 = pl.pallas_call(
        _mm2_body,
        grid=(grid,),
        in_specs=[_pair_spec(D_HID), _row_spec(D_HID), _col_spec(),
                  _full_spec(1, D_HID), _full_spec(D_HID, D_HID)],
        out_specs=_row_spec(D_HID),
        out_shape=jax.ShapeDtypeStruct((NP, D_HID), jnp.float32),
    )(s_pair, g1, dinv, b1r, Wcat)

    t_pair = _scatter_kernel(g2, ep).reshape(NC, NP, D_HID)

    out = pl.pallas_call(
        _fin_body,
        grid=(grid,),
        in_specs=[_pair_spec(D_HID), _row_spec(D_HID), _col_spec(),
                  _full_spec(1, D_HID)],
        out_specs=_row_spec(D_HID),
        out_shape=jax.ShapeDtypeStruct((NP, D_HID), jnp.float32),
    )(t_pair, g2, dinv, bcat)

    out = out[:N]
    return (out[:, :D_OUT], out[:, D_OUT:])
